# baseline (speedup 1.0000x reference)
import functools

import jax
import jax.numpy as jnp
from jax import lax
from jax.experimental import pallas as pl
from jax.experimental.pallas import tpu as pltpu
from jax.experimental.pallas import tpu_sc as plsc

N = 10000
E = 320000
D = 128

NC = 2
NS = 16
NW = NC * NS
CHUNK = 128
G = 8
GROUPS = 10
C_PER_TILE = G * GROUPS
E_PAD = NW * C_PER_TILE * CHUNK
N_PAD = 10112
ROWS_PER_TILE = N_PAD // NS
DEG_W = 16


def _lin_body(x_ref, wl_ref, wr_ref, b_ref, z_ref, w_ref):
    x = x_ref[...]
    z_ref[...] = lax.dot_general(x, wl_ref[...], (((1,), (1,)), ((), ())),
                                 preferred_element_type=jnp.float32)
    w_ref[...] = lax.dot_general(x, wr_ref[...], (((1,), (1,)), ((), ())),
                                 preferred_element_type=jnp.float32) + b_ref[...]


def _combine_body(acc_ref, deg_ref, w_ref, o_ref):
    a = acc_ref[0, :N, :] + acc_ref[1, :N, :]
    d = deg_ref[0, :N, 0:1] + deg_ref[1, :N, 0:1]
    o_ref[...] = a / jnp.maximum(d, 1.0) + w_ref[...]


_mesh = plsc.VectorSubcoreMesh(core_axis_name="c", subcore_axis_name="s")


@functools.partial(
    pl.kernel,
    mesh=_mesh,
    compiler_params=pltpu.CompilerParams(use_tc_tiling_on_sc=False),
    out_type=[
        jax.ShapeDtypeStruct((NC, N_PAD, D), jnp.float32),
        jax.ShapeDtypeStruct((NC, N_PAD, DEG_W), jnp.float32),
    ],
    scratch_types=[
        pltpu.VMEM((2, G, CHUNK), jnp.int32),
        pltpu.VMEM((2, G, CHUNK), jnp.int32),
        pltpu.VMEM((2, CHUNK, D), jnp.float32),
        pltpu.VMEM((CHUNK, DEG_W), jnp.float32),
        pltpu.VMEM_SHARED((N_PAD, D), jnp.float32),
        pltpu.VMEM_SHARED((N_PAD, DEG_W), jnp.float32),
        pltpu.SemaphoreType.DMA,
        pltpu.SemaphoreType.DMA,
        pltpu.SemaphoreType.DMA,
        pltpu.SemaphoreType.DMA,
        pltpu.SemaphoreType.DMA,
        pltpu.SemaphoreType.DMA,
    ],
)
def _sc_agg(z_hbm, src_hbm, dst_hbm, zeros_hbm, zeros8_hbm, ones_hbm,
            acc_out, deg_out, src_v, dst_v, rows2, ones_v, acc_sh, deg_sh,
            sem_g0, sem_g1, sem_s0, sem_s1, sem_i, sem_d):
    c = lax.axis_index("c")
    s = lax.axis_index("s")
    w = c * NS + s
    r0 = s * ROWS_PER_TILE
    pieces = [(k * CHUNK, min(CHUNK, ROWS_PER_TILE - k * CHUNK))
              for k in range(-(-ROWS_PER_TILE // CHUNK))]

    pltpu.sync_copy(zeros_hbm, rows2.at[0])
    pltpu.sync_copy(zeros8_hbm, ones_v)
    for off, sz in pieces:
        pltpu.sync_copy(rows2.at[0].at[pl.ds(0, sz)],
                        acc_sh.at[pl.ds(r0 + off, sz)])
        pltpu.sync_copy(ones_v.at[pl.ds(0, sz)], deg_sh.at[pl.ds(r0 + off, sz)])
    pltpu.sync_copy(ones_hbm, ones_v)
    plsc.subcore_barrier()

    sem_g = (sem_g0, sem_g1)
    sem_s = (sem_s0, sem_s1)

    base0 = w * C_PER_TILE
    pltpu.async_copy(src_hbm.at[pl.ds(base0, G)], src_v.at[0], sem_i)
    pltpu.async_copy(dst_hbm.at[pl.ds(base0, G)], dst_v.at[0], sem_i)

    def body(g, carry):
        slot = lax.rem(g, 2)
        pltpu.make_async_copy(src_hbm.at[pl.ds(0, G)], src_v.at[slot],
                              sem_i).wait()
        pltpu.make_async_copy(dst_hbm.at[pl.ds(0, G)], dst_v.at[slot],
                              sem_i).wait()
        @pl.when(g + 1 < GROUPS)
        def _():
            nbase = w * C_PER_TILE + (g + 1) * G
            pltpu.async_copy(src_hbm.at[pl.ds(nbase, G)], src_v.at[1 - slot],
                             sem_i)
            pltpu.async_copy(dst_hbm.at[pl.ds(nbase, G)], dst_v.at[1 - slot],
                             sem_i)

        sv = src_v.at[slot]
        dv = dst_v.at[slot]
        hg = [None] * G
        hs = [None] * G
        hg[0] = pltpu.async_copy(z_hbm.at[sv.at[0]], rows2.at[0], sem_g[0])
        for j in range(G):
            b = j % 2
            if j + 1 < G:
                if j - 1 >= 0:
                    hs[j - 1].wait()
                hg[j + 1] = pltpu.async_copy(z_hbm.at[sv.at[j + 1]],
                                             rows2.at[1 - b], sem_g[1 - b])
            hg[j].wait()
            hs[j] = pltpu.async_copy(rows2.at[b], acc_sh.at[dv.at[j]],
                                     sem_s[b], add=True)
            pltpu.async_copy(ones_v, deg_sh.at[dv.at[j]], sem_d, add=True)
        hs[G - 2].wait()
        hs[G - 1].wait()
        for j in range(G):
            pltpu.make_async_copy(ones_hbm, ones_v, sem_d).wait()
        return carry

    lax.fori_loop(0, GROUPS, body, 0)
    plsc.subcore_barrier()

    for off, sz in pieces:
        pltpu.sync_copy(acc_sh.at[pl.ds(r0 + off, sz)],
                        rows2.at[0].at[pl.ds(0, sz)])
        pltpu.sync_copy(rows2.at[0].at[pl.ds(0, sz)],
                        acc_out.at[c].at[pl.ds(r0 + off, sz)])
        pltpu.sync_copy(deg_sh.at[pl.ds(r0 + off, sz)], ones_v.at[pl.ds(0, sz)])
        pltpu.sync_copy(ones_v.at[pl.ds(0, sz)],
                        deg_out.at[c].at[pl.ds(r0 + off, sz)])


def kernel(x, edge_index, W_l, b_l, W_r):
    ei = edge_index.astype(jnp.int32)
    pad = E_PAD - E
    src2 = jnp.concatenate([ei[0], jnp.zeros((pad,), jnp.int32)]
                           ).reshape(NW * C_PER_TILE, CHUNK)
    dst2 = jnp.concatenate([ei[1], jnp.full((pad,), N, jnp.int32)]
                           ).reshape(NW * C_PER_TILE, CHUNK)

    z, w = pl.pallas_call(
        _lin_body,
        out_shape=[jax.ShapeDtypeStruct((N, D), jnp.float32),
                   jax.ShapeDtypeStruct((N, D), jnp.float32)],
    )(x, W_l, W_r, b_l.reshape(1, D))

    zeros = jnp.zeros((CHUNK, D), jnp.float32)
    zeros8 = jnp.zeros((CHUNK, DEG_W), jnp.float32)
    ones = jnp.ones((CHUNK, DEG_W), jnp.float32)
    acc, deg = _sc_agg(z, src2, dst2, zeros, zeros8, ones)

    out = pl.pallas_call(
        _combine_body,
        out_shape=jax.ShapeDtypeStruct((N, D), jnp.float32),
    )(acc, deg, w)
    return out


# speedup vs baseline: 1.2456x; 1.2456x over previous
import functools

import jax
import jax.numpy as jnp
from jax import lax
from jax.experimental import pallas as pl
from jax.experimental.pallas import tpu as pltpu
from jax.experimental.pallas import tpu_sc as plsc

N = 10000
E = 320000
D = 128

NC = 2
NS = 16
NW = NC * NS
CHUNK = 64
G = 8
GROUPS = 20
C_PER_TILE = G * GROUPS
E_PAD = NW * C_PER_TILE * CHUNK
N_PAD = 10112
ROWS_PER_TILE = N_PAD // NS
DEG_W = 16


def _lin_body(x_ref, wl_ref, wr_ref, b_ref, z_ref, w_ref):
    x = x_ref[...]
    z_ref[...] = lax.dot_general(x, wl_ref[...], (((1,), (1,)), ((), ())),
                                 preferred_element_type=jnp.float32)
    w_ref[...] = lax.dot_general(x, wr_ref[...], (((1,), (1,)), ((), ())),
                                 preferred_element_type=jnp.float32) + b_ref[...]


def _combine_body(acc_ref, deg_ref, w_ref, o_ref):
    a = acc_ref[0, :N, :] + acc_ref[1, :N, :]
    d = deg_ref[0, :N, 0:1] + deg_ref[1, :N, 0:1]
    o_ref[...] = a / jnp.maximum(d, 1.0) + w_ref[...]


_mesh = plsc.VectorSubcoreMesh(core_axis_name="c", subcore_axis_name="s")


@functools.partial(
    pl.kernel,
    mesh=_mesh,
    compiler_params=pltpu.CompilerParams(use_tc_tiling_on_sc=False,
                                         needs_layout_passes=False),
    out_type=[
        jax.ShapeDtypeStruct((NC, N_PAD, D), jnp.float32),
        jax.ShapeDtypeStruct((NC, N_PAD, DEG_W), jnp.float32),
    ],
    scratch_types=[
        pltpu.VMEM((G, CHUNK), jnp.int32),
        pltpu.VMEM((G, CHUNK), jnp.int32),
        pltpu.VMEM((2, CHUNK, D), jnp.bfloat16),
        pltpu.VMEM((2, CHUNK, D), jnp.float32),
        pltpu.VMEM((CHUNK, DEG_W), jnp.float32),
        pltpu.VMEM_SHARED((N_PAD, D), jnp.float32),
        pltpu.VMEM_SHARED((N_PAD, DEG_W), jnp.float32),
        pltpu.SemaphoreType.DMA,
        pltpu.SemaphoreType.DMA,
        pltpu.SemaphoreType.DMA,
        pltpu.SemaphoreType.DMA,
    ],
)
def _sc_agg(zb_hbm, src_hbm, dst_hbm, zeros_hbm, zeros8_hbm, ones_hbm,
            acc_out, deg_out, src_v, dst_v, rows_bf, rows_f, ones_v,
            acc_sh, deg_sh, sem_g0, sem_g1, sem_s0, sem_s1):
    c = lax.axis_index("c")
    s = lax.axis_index("s")
    w = c * NS + s
    r0 = s * ROWS_PER_TILE
    pieces = [(k * CHUNK, min(CHUNK, ROWS_PER_TILE - k * CHUNK))
              for k in range(-(-ROWS_PER_TILE // CHUNK))]

    pltpu.sync_copy(zeros_hbm, rows_f.at[0])
    pltpu.sync_copy(zeros8_hbm, ones_v)
    for off, sz in pieces:
        pltpu.sync_copy(rows_f.at[0].at[pl.ds(0, sz)],
                        acc_sh.at[pl.ds(r0 + off, sz)])
        pltpu.sync_copy(ones_v.at[pl.ds(0, sz)], deg_sh.at[pl.ds(r0 + off, sz)])
    pltpu.sync_copy(ones_hbm, ones_v)
    plsc.subcore_barrier()

    sem_g = (sem_g0, sem_g1)
    sem_s = (sem_s0, sem_s1)

    def unpack_chunk(b):
        def row(r, carry):
            for blk in range(D // 32):
                ab = rows_bf.at[b].at[r][pl.ds(blk * 32, 32)]
                lo, hi = plsc.unpack(ab, format=plsc.PackFormat.INTERLEAVED)
                rows_f.at[b].at[r][pl.ds(blk * 32, 16)] = lo
                rows_f.at[b].at[r][pl.ds(blk * 32 + 16, 16)] = hi
            return carry
        lax.fori_loop(0, CHUNK, row, 0)

    def body(g, carry):
        base = w * C_PER_TILE + g * G
        pltpu.sync_copy(src_hbm.at[pl.ds(base, G)], src_v)
        pltpu.sync_copy(dst_hbm.at[pl.ds(base, G)], dst_v)
        hg = [None] * G
        hs = [None] * G
        hg[0] = pltpu.async_copy(zb_hbm.at[src_v.at[0]], rows_bf.at[0],
                                 sem_g[0])
        for j in range(G):
            b = j % 2
            if j + 1 < G:
                hg[j + 1] = pltpu.async_copy(zb_hbm.at[src_v.at[j + 1]],
                                             rows_bf.at[1 - b], sem_g[1 - b])
            hg[j].wait()
            if j - 2 >= 0:
                hs[j - 2].wait()
            unpack_chunk(b)
            hs[j] = pltpu.async_copy(rows_f.at[b], acc_sh.at[dst_v.at[j]],
                                     sem_s[b], add=True)
            pltpu.sync_copy(ones_v, deg_sh.at[dst_v.at[j]], add=True)
        hs[G - 2].wait()
        hs[G - 1].wait()
        return carry

    lax.fori_loop(0, GROUPS, body, 0)
    plsc.subcore_barrier()

    for off, sz in pieces:
        pltpu.sync_copy(acc_sh.at[pl.ds(r0 + off, sz)],
                        rows_f.at[0].at[pl.ds(0, sz)])
        pltpu.sync_copy(rows_f.at[0].at[pl.ds(0, sz)],
                        acc_out.at[c].at[pl.ds(r0 + off, sz)])
        pltpu.sync_copy(deg_sh.at[pl.ds(r0 + off, sz)], ones_v.at[pl.ds(0, sz)])
        pltpu.sync_copy(ones_v.at[pl.ds(0, sz)],
                        deg_out.at[c].at[pl.ds(r0 + off, sz)])


def kernel(x, edge_index, W_l, b_l, W_r):
    ei = edge_index.astype(jnp.int32)
    pad = E_PAD - E
    src2 = jnp.concatenate([ei[0], jnp.zeros((pad,), jnp.int32)]
                           ).reshape(NW * C_PER_TILE, CHUNK)
    dst2 = jnp.concatenate([ei[1], jnp.full((pad,), N, jnp.int32)]
                           ).reshape(NW * C_PER_TILE, CHUNK)

    z, w = pl.pallas_call(
        _lin_body,
        out_shape=[jax.ShapeDtypeStruct((N, D), jnp.float32),
                   jax.ShapeDtypeStruct((N, D), jnp.float32)],
    )(x, W_l, W_r, b_l.reshape(1, D))

    zb = (z.reshape(N, D // 32, 2, 16).transpose(0, 1, 3, 2).reshape(N, D)
          .astype(jnp.bfloat16))

    zeros = jnp.zeros((CHUNK, D), jnp.float32)
    zeros8 = jnp.zeros((CHUNK, DEG_W), jnp.float32)
    ones = jnp.ones((CHUNK, DEG_W), jnp.float32)
    acc, deg = _sc_agg(zb, src2, dst2, zeros, zeros8, ones)

    out = pl.pallas_call(
        _combine_body,
        out_shape=jax.ShapeDtypeStruct((N, D), jnp.float32),
    )(acc, deg, w)
    return out
